# baseline (speedup 1.0000x reference)
import dataclasses
import functools

import jax
import jax.numpy as jnp
from jax.experimental import pallas as pl
from jax.experimental.pallas import tpu as pltpu
from jax.experimental.pallas import tpu_sc as plsc

B = 16384
D = 64
K = 50
NNEG = 16
T = 5
M = 1.0
LAMBDA = 0.0001

BT = 2048
NCHUNK = 1
BC = B // NCHUNK
NROW = 16
RG = 2

_SC_MESH = plsc.VectorSubcoreMesh(core_axis_name="c", subcore_axis_name="s")

_SC_PARAMS = pltpu.CompilerParams()
if "needs_layout_passes" in pltpu.CompilerParams.__dataclass_fields__:
    _SC_PARAMS = dataclasses.replace(_SC_PARAMS, needs_layout_passes=False)


def _sc_top5(gp, ga0, ga1, gfx, off):
    out_t = [jax.ShapeDtypeStruct((BC,), jnp.float32)] * 8
    off_blk = off // (RG * NROW)

    @functools.partial(pl.kernel, out_type=out_t, mesh=_SC_MESH,
                       compiler_params=_SC_PARAMS)
    def sck(gp_h, ga0_h, ga1_h, gfx_h,
            tp_h, ta0_h, ta1_h, tfx_h,
            ip_h, ia0_h, ia1_h, ifx_h):
        def body(gp_v, ga0_v, ga1_v, gfx_v,
                 tp_v, ta0_v, ta1_v, tfx_v,
                 ip_v, ia0_v, ia1_v, ifx_v):
            base = jax.lax.iota(jnp.int32, NROW)
            zero = jnp.zeros((NROW,), jnp.int32)
            inf = jnp.full((NROW,), jnp.inf, dtype=jnp.float32)
            for g_v, t_v, i_v in ((gp_v, tp_v, ip_v), (ga0_v, ta0_v, ia0_v),
                                  (ga1_v, ta1_v, ia1_v),
                                  (gfx_v, tfx_v, ifx_v)):
                for rg in range(RG):
                    rows = base + (rg * NROW)
                    m0 = m1 = m2 = m3 = m4 = inf
                    for k in range(K):
                        e = plsc.load_gather(g_v, [rows, zero + k])
                        lo = jnp.minimum(m0, e); e = jnp.maximum(m0, e); m0 = lo
                        lo = jnp.minimum(m1, e); e = jnp.maximum(m1, e); m1 = lo
                        lo = jnp.minimum(m2, e); e = jnp.maximum(m2, e); m2 = lo
                        lo = jnp.minimum(m3, e); e = jnp.maximum(m3, e); m3 = lo
                        m4 = jnp.minimum(m4, e)
                    s = ((m0 + m1) + (m2 + m3)) + m4
                    sl = pl.ds(rg * NROW, NROW)
                    t_v[sl] = m4
                    i_v[sl] = jnp.where(s > 0.0, 1.0 / s, 0.0)

        pltpu.emit_pipeline(
            body,
            grid=(BC // (RG * NROW),),
            in_specs=[pl.BlockSpec((RG * NROW, K),
                                   lambda i, o=off_blk: (i + o, 0))] * 4,
            out_specs=[pl.BlockSpec((RG * NROW,), lambda i: (i,))] * 8,
            core_axis_name=("c", "s"),
            dimension_semantics=(pltpu.PARALLEL,),
        )(gp_h, ga0_h, ga1_h, gfx_h, tp_h, ta0_h, ta1_h, tfx_h,
          ip_h, ia0_h, ia1_h, ifx_h)

    return sck(gp, ga0, ga1, gfx)


def _dot(a, b):
    return jax.lax.dot_general(
        a.astype(jnp.bfloat16), b.astype(jnp.bfloat16),
        (((1,), (1,)), ((), ())),
        preferred_element_type=jnp.float32,
    )


def _ortho_kernel(F_p, F_a0, F_a1, F_fx, out_ref):
    total = 0.0
    for f_ref in (F_p, F_a0, F_a1, F_fx):
        F = f_ref[...]
        gram = _dot(F, F)
        ii = jax.lax.broadcasted_iota(jnp.int32, gram.shape, 0)
        jj = jax.lax.broadcasted_iota(jnp.int32, gram.shape, 1)
        eye = (ii == jj).astype(jnp.float32)
        s = jnp.sum(jnp.abs(gram - eye))
        total += (LAMBDA * s) * s
    out_ref[...] = jnp.reshape(total, (1, 1))


def _group_loss_tile(v, vhat, g, theta, inv, F, negs):
    diff = vhat - v
    true_d = jnp.sqrt(jnp.sum(diff * diff, axis=1, keepdims=True))
    vhat_sq = jnp.sum(vhat * vhat, axis=1, keepdims=True)

    w = jnp.concatenate([negs, F], axis=0)
    wsq = jnp.sum(w * w, axis=1)[None, :]
    dotw = _dot(vhat, w)
    dist = jnp.sqrt(jnp.maximum(vhat_sq - 2.0 * dotw + wsq, 0.0))

    mask5 = g <= theta
    gt = g * inv
    mt = M * (1.0 - gt) ** 2
    margin = jnp.concatenate(
        [jnp.ones((g.shape[0], NNEG), jnp.float32), mt], axis=1)
    wgt = jnp.concatenate(
        [jnp.full((g.shape[0], NNEG), 1.0 / NNEG, jnp.float32),
         jnp.where(mask5, 1.0 / T, 0.0)], axis=1)
    term = jnp.maximum(margin + true_d - dist, 0.0)
    return jnp.sum(term * wgt, axis=1, keepdims=True)


def _loss_kernel(ortho, v_p, vh_p, g_p, t_p, i_p, F_p, n_p,
                 v_a0, vh_a0, g_a0, t_a0, i_a0, F_a0, n_a0,
                 v_a1, vh_a1, g_a1, t_a1, i_a1, F_a1, n_a1,
                 v_fx, vh_fx, g_fx, t_fx, i_fx, F_fx, n_fx,
                 out_ref):
    acc = _group_loss_tile(v_p[...], vh_p[...], g_p[...], t_p[...], i_p[...],
                           F_p[...], n_p[...])
    acc += _group_loss_tile(v_a0[...], vh_a0[...], g_a0[...], t_a0[...],
                            i_a0[...], F_a0[...], n_a0[...])
    acc += _group_loss_tile(v_a1[...], vh_a1[...], g_a1[...], t_a1[...],
                            i_a1[...], F_a1[...], n_a1[...])
    acc += _group_loss_tile(v_fx[...], vh_fx[...], g_fx[...], t_fx[...],
                            i_fx[...], F_fx[...], n_fx[...])
    out_ref[...] = acc + ortho[...]


def _tc_chunk(ortho, groups, sc_outs, off):
    off_blk = off // BT
    bspec_vd = pl.BlockSpec((BT, D), lambda i, o=off_blk: (i + o, 0))
    bspec_g = pl.BlockSpec((BT, K), lambda i, o=off_blk: (i + o, 0))
    bspec_c = pl.BlockSpec((BT, 1), lambda i: (i, 0))
    bspec_F = pl.BlockSpec((K, D), lambda i: (0, 0))
    bspec_n = pl.BlockSpec((NNEG, D), lambda i: (0, 0))
    in_specs = [pl.BlockSpec((1, 1), lambda i: (0, 0))]
    args = [ortho]
    for gi, (v, vh, g, F, n) in enumerate(groups):
        theta, inv = sc_outs[gi], sc_outs[4 + gi]
        in_specs += [bspec_vd, bspec_vd, bspec_g, bspec_c, bspec_c,
                     bspec_F, bspec_n]
        args += [v, vh, g, theta.reshape(BC, 1), inv.reshape(BC, 1), F, n]
    return pl.pallas_call(
        _loss_kernel,
        grid=(BC // BT,),
        in_specs=in_specs,
        out_specs=pl.BlockSpec((BT, 1), lambda i: (i, 0)),
        out_shape=jax.ShapeDtypeStruct((BC, 1), jnp.float32),
        compiler_params=pltpu.CompilerParams(
            dimension_semantics=("parallel",)),
    )(*args)


@jax.jit
def _run(groups):
    ortho = pl.pallas_call(
        _ortho_kernel,
        out_shape=jax.ShapeDtypeStruct((1, 1), jnp.float32),
    )(*[gr[3] for gr in groups])

    gs = [gr[2] for gr in groups]
    sc_outs = [_sc_top5(*gs, off=c * BC) for c in range(NCHUNK)]
    chunks = [_tc_chunk(ortho, groups, sc_outs[c], off=c * BC)
              for c in range(NCHUNK)]
    return jnp.concatenate(chunks, axis=0).reshape(B)


def kernel(v_p, vhat_p, d_p, g_p, F_p,
           v_a0, vhat_a0, d_a0, g_a0, F_a0,
           v_a1, vhat_a1, d_a1, g_a1, F_a1,
           v_fx, vhat_fx, d_fx, g_fx, F_fx,
           p_negatives, a0_negatives, a1_negatives, fx_negatives):
    groups = [
        (v_p, vhat_p, g_p, F_p, p_negatives),
        (v_a0, vhat_a0, g_a0, F_a0, a0_negatives),
        (v_a1, vhat_a1, g_a1, F_a1, a1_negatives),
        (v_fx, vhat_fx, g_fx, F_fx, fx_negatives),
    ]
    return _run(groups)


# speedup vs baseline: 1.0542x; 1.0542x over previous
import dataclasses
import functools

import jax
import jax.numpy as jnp
from jax.experimental import pallas as pl
from jax.experimental.pallas import tpu as pltpu
from jax.experimental.pallas import tpu_sc as plsc

B = 16384
D = 64
K = 50
NNEG = 16
T = 5
M = 1.0
LAMBDA = 0.0001

BT = 2048
NCHUNK = 1
BC = B // NCHUNK
NROW = 16
RG = 1

_SC_MESH = plsc.VectorSubcoreMesh(core_axis_name="c", subcore_axis_name="s")

_SC_PARAMS = pltpu.CompilerParams()
if "needs_layout_passes" in pltpu.CompilerParams.__dataclass_fields__:
    _SC_PARAMS = dataclasses.replace(_SC_PARAMS, needs_layout_passes=False)


def _sc_top5(gp, ga0, ga1, gfx, off):
    out_t = [jax.ShapeDtypeStruct((BC,), jnp.float32)] * 8
    off_blk = off // (RG * NROW)

    @functools.partial(pl.kernel, out_type=out_t, mesh=_SC_MESH,
                       compiler_params=_SC_PARAMS)
    def sck(gp_h, ga0_h, ga1_h, gfx_h,
            tp_h, ta0_h, ta1_h, tfx_h,
            ip_h, ia0_h, ia1_h, ifx_h):
        def body(gp_v, ga0_v, ga1_v, gfx_v,
                 tp_v, ta0_v, ta1_v, tfx_v,
                 ip_v, ia0_v, ia1_v, ifx_v):
            base = jax.lax.iota(jnp.int32, NROW)
            zero = jnp.zeros((NROW,), jnp.int32)
            inf = jnp.full((NROW,), jnp.inf, dtype=jnp.float32)
            for g_v, t_v, i_v in ((gp_v, tp_v, ip_v), (ga0_v, ta0_v, ia0_v),
                                  (ga1_v, ta1_v, ia1_v),
                                  (gfx_v, tfx_v, ifx_v)):
                for rg in range(RG):
                    rows = base + (rg * NROW)
                    m0 = m1 = m2 = m3 = m4 = inf
                    for k in range(K):
                        e = plsc.load_gather(g_v, [rows, zero + k])
                        lo = jnp.minimum(m0, e); e = jnp.maximum(m0, e); m0 = lo
                        lo = jnp.minimum(m1, e); e = jnp.maximum(m1, e); m1 = lo
                        lo = jnp.minimum(m2, e); e = jnp.maximum(m2, e); m2 = lo
                        lo = jnp.minimum(m3, e); e = jnp.maximum(m3, e); m3 = lo
                        m4 = jnp.minimum(m4, e)
                    s = ((m0 + m1) + (m2 + m3)) + m4
                    sl = pl.ds(rg * NROW, NROW)
                    t_v[sl] = m4
                    i_v[sl] = jnp.where(s > 0.0, 1.0 / s, 0.0)

        pltpu.emit_pipeline(
            body,
            grid=(BC // (RG * NROW),),
            in_specs=[pl.BlockSpec((RG * NROW, K),
                                   lambda i, o=off_blk: (i + o, 0))] * 4,
            out_specs=[pl.BlockSpec((RG * NROW,), lambda i: (i,))] * 8,
            core_axis_name=("c", "s"),
            dimension_semantics=(pltpu.PARALLEL,),
        )(gp_h, ga0_h, ga1_h, gfx_h, tp_h, ta0_h, ta1_h, tfx_h,
          ip_h, ia0_h, ia1_h, ifx_h)

    return sck(gp, ga0, ga1, gfx)


def _dot(a, b):
    return jax.lax.dot_general(
        a.astype(jnp.bfloat16), b.astype(jnp.bfloat16),
        (((1,), (1,)), ((), ())),
        preferred_element_type=jnp.float32,
    )


def _ortho_kernel(F_p, F_a0, F_a1, F_fx, out_ref):
    total = 0.0
    for f_ref in (F_p, F_a0, F_a1, F_fx):
        F = f_ref[...]
        gram = _dot(F, F)
        ii = jax.lax.broadcasted_iota(jnp.int32, gram.shape, 0)
        jj = jax.lax.broadcasted_iota(jnp.int32, gram.shape, 1)
        eye = (ii == jj).astype(jnp.float32)
        s = jnp.sum(jnp.abs(gram - eye))
        total += (LAMBDA * s) * s
    out_ref[...] = jnp.reshape(total, (1, 1))


def _group_loss_tile(v, vhat, g, theta, inv, F, negs):
    diff = vhat - v
    true_d = jnp.sqrt(jnp.sum(diff * diff, axis=1, keepdims=True))
    vhat_sq = jnp.sum(vhat * vhat, axis=1, keepdims=True)

    w = jnp.concatenate([negs, F], axis=0)
    wsq = jnp.sum(w * w, axis=1)[None, :]
    dotw = _dot(vhat, w)
    dist = jnp.sqrt(jnp.maximum(vhat_sq - 2.0 * dotw + wsq, 0.0))

    mask5 = g <= theta
    gt = g * inv
    mt = M * (1.0 - gt) ** 2
    margin = jnp.concatenate(
        [jnp.ones((g.shape[0], NNEG), jnp.float32), mt], axis=1)
    wgt = jnp.concatenate(
        [jnp.full((g.shape[0], NNEG), 1.0 / NNEG, jnp.float32),
         jnp.where(mask5, 1.0 / T, 0.0)], axis=1)
    term = jnp.maximum(margin + true_d - dist, 0.0)
    return jnp.sum(term * wgt, axis=1, keepdims=True)


def _loss_kernel(ortho, v_p, vh_p, g_p, t_p, i_p, F_p, n_p,
                 v_a0, vh_a0, g_a0, t_a0, i_a0, F_a0, n_a0,
                 v_a1, vh_a1, g_a1, t_a1, i_a1, F_a1, n_a1,
                 v_fx, vh_fx, g_fx, t_fx, i_fx, F_fx, n_fx,
                 out_ref):
    acc = _group_loss_tile(v_p[...], vh_p[...], g_p[...], t_p[...], i_p[...],
                           F_p[...], n_p[...])
    acc += _group_loss_tile(v_a0[...], vh_a0[...], g_a0[...], t_a0[...],
                            i_a0[...], F_a0[...], n_a0[...])
    acc += _group_loss_tile(v_a1[...], vh_a1[...], g_a1[...], t_a1[...],
                            i_a1[...], F_a1[...], n_a1[...])
    acc += _group_loss_tile(v_fx[...], vh_fx[...], g_fx[...], t_fx[...],
                            i_fx[...], F_fx[...], n_fx[...])
    out_ref[...] = acc + ortho[...]


def _tc_chunk(ortho, groups, sc_outs, off):
    off_blk = off // BT
    bspec_vd = pl.BlockSpec((BT, D), lambda i, o=off_blk: (i + o, 0))
    bspec_g = pl.BlockSpec((BT, K), lambda i, o=off_blk: (i + o, 0))
    bspec_c = pl.BlockSpec((BT, 1), lambda i: (i, 0))
    bspec_F = pl.BlockSpec((K, D), lambda i: (0, 0))
    bspec_n = pl.BlockSpec((NNEG, D), lambda i: (0, 0))
    in_specs = [pl.BlockSpec((1, 1), lambda i: (0, 0))]
    args = [ortho]
    for gi, (v, vh, g, F, n) in enumerate(groups):
        theta, inv = sc_outs[gi], sc_outs[4 + gi]
        in_specs += [bspec_vd, bspec_vd, bspec_g, bspec_c, bspec_c,
                     bspec_F, bspec_n]
        args += [v, vh, g, theta.reshape(BC, 1), inv.reshape(BC, 1), F, n]
    return pl.pallas_call(
        _loss_kernel,
        grid=(BC // BT,),
        in_specs=in_specs,
        out_specs=pl.BlockSpec((BT, 1), lambda i: (i, 0)),
        out_shape=jax.ShapeDtypeStruct((BC, 1), jnp.float32),
        compiler_params=pltpu.CompilerParams(
            dimension_semantics=("parallel",)),
    )(*args)


@jax.jit
def _run(groups):
    ortho = pl.pallas_call(
        _ortho_kernel,
        out_shape=jax.ShapeDtypeStruct((1, 1), jnp.float32),
    )(*[gr[3] for gr in groups])

    gs = [gr[2] for gr in groups]
    sc_outs = [_sc_top5(*gs, off=c * BC) for c in range(NCHUNK)]
    chunks = [_tc_chunk(ortho, groups, sc_outs[c], off=c * BC)
              for c in range(NCHUNK)]
    return jnp.concatenate(chunks, axis=0).reshape(B)


def kernel(v_p, vhat_p, d_p, g_p, F_p,
           v_a0, vhat_a0, d_a0, g_a0, F_a0,
           v_a1, vhat_a1, d_a1, g_a1, F_a1,
           v_fx, vhat_fx, d_fx, g_fx, F_fx,
           p_negatives, a0_negatives, a1_negatives, fx_negatives):
    groups = [
        (v_p, vhat_p, g_p, F_p, p_negatives),
        (v_a0, vhat_a0, g_a0, F_a0, a0_negatives),
        (v_a1, vhat_a1, g_a1, F_a1, a1_negatives),
        (v_fx, vhat_fx, g_fx, F_fx, fx_negatives),
    ]
    return _run(groups)
